# Initial kernel scaffold; baseline (speedup 1.0000x reference)
import jax
import jax.numpy as jnp
from jax.experimental import pallas as pl


def kernel(x, edge_index, edge_attr, emb, W1, b1, W2, b2):
    raise NotImplementedError("write your pallas kernel here")


# speedup vs baseline: 8.5659x; 8.5659x over previous
import functools

import jax
import jax.numpy as jnp
from jax import lax
from jax.experimental import pallas as pl
from jax.experimental.pallas import tpu as pltpu
from jax.experimental.pallas import tpu_sc as plsc

CH = 128
XCH = 64

_SC_PARAMS = pltpu.CompilerParams(needs_layout_passes=False,
                                  use_tc_tiling_on_sc=False)


def _sc_mesh():
    return plsc.VectorSubcoreMesh(core_axis_name="c", subcore_axis_name="s")


def _emb_deg_kernel(V, D, N_pad, NCH, NC, NS):
    NW = NC * NS
    rows_w = N_pad // NW
    nx = rows_w // XCH
    rows_s = N_pad // NS

    @functools.partial(
        pl.kernel,
        out_type=[
            jax.ShapeDtypeStruct((N_pad, D), jnp.float32),
            jax.ShapeDtypeStruct((NC, N_pad, 16), jnp.float32),
        ],
        mesh=_sc_mesh(),
        compiler_params=_SC_PARAMS,
        scratch_types=[
            pltpu.VMEM((nx, XCH), jnp.int32),
            pltpu.VMEM((XCH, D), jnp.float32),
            pltpu.VMEM((NCH, CH), jnp.int32),
            pltpu.VMEM((CH, 16), jnp.float32),
            pltpu.VMEM_SHARED((N_pad, 16), jnp.float32),
        ],
    )
    def k(emb_h, xi_h, col_h, wb_h, z16_h, h0_h, deg_h,
          xi_v, rows_v, col_v, wrow_v, deg_sh):
        c = lax.axis_index("c")
        s = lax.axis_index("s")
        wid = s * NC + c
        sl = pl.ds(s * rows_s, rows_s)
        pltpu.sync_copy(z16_h.at[sl], deg_sh.at[sl])
        pltpu.sync_copy(xi_h.at[wid], xi_v)
        for j in range(nx):
            pltpu.sync_copy(emb_h.at[xi_v.at[j]], rows_v)
            pltpu.sync_copy(rows_v, h0_h.at[pl.ds(wid * rows_w + j * XCH, XCH)])
        pltpu.sync_copy(col_h.at[wid], col_v)
        plsc.subcore_barrier()

        def chunk(j, carry):
            pltpu.sync_copy(wb_h.at[wid].at[j], wrow_v)
            pltpu.sync_copy(wrow_v, deg_sh.at[col_v.at[j]], add=True)
            return carry

        lax.fori_loop(0, NCH, chunk, 0)
        plsc.subcore_barrier()
        pltpu.sync_copy(deg_sh.at[sl], deg_h.at[c].at[sl])

    return k


def _agg_kernel(Dc, N_pad, NCH, NC, NS):
    NW = NC * NS
    ECH = NCH * CH
    rows_s = N_pad // NS

    @functools.partial(
        pl.kernel,
        out_type=jax.ShapeDtypeStruct((NC, N_pad, Dc), jnp.float32),
        mesh=_sc_mesh(),
        compiler_params=_SC_PARAMS,
        scratch_types=[
            pltpu.VMEM((NCH, CH), jnp.int32),
            pltpu.VMEM((NCH, CH), jnp.int32),
            pltpu.VMEM((ECH,), jnp.float32),
            pltpu.VMEM((CH, Dc), jnp.float32),
            pltpu.VMEM_SHARED((N_pad, Dc), jnp.float32),
        ],
    )
    def k(g_h, row_h, col_h, w_h, acc_h, row_v, col_v, w_v, rows_v, acc_sh):
        c = lax.axis_index("c")
        s = lax.axis_index("s")
        wid = s * NC + c
        sl = pl.ds(s * rows_s, rows_s)
        pltpu.sync_copy(g_h.at[sl], acc_sh.at[sl])
        pltpu.sync_copy(row_h.at[wid], row_v)
        pltpu.sync_copy(col_h.at[wid], col_v)
        pltpu.sync_copy(w_h.at[wid], w_v)
        plsc.subcore_barrier()

        def chunk(j, carry):
            pltpu.sync_copy(g_h.at[row_v.at[j]], rows_v)

            def scale(i, carry2):
                idx = lax.broadcast_in_dim(j * CH + i, (16,), ())
                wb = plsc.load_gather(w_v, [idx])
                for kk in range(Dc // 16):
                    csl = pl.ds(kk * 16, 16)
                    rows_v[i, csl] = rows_v[i, csl] * wb
                return carry2

            lax.fori_loop(0, CH, scale, 0, unroll=2)
            pltpu.sync_copy(rows_v, acc_sh.at[col_v.at[j]], add=True)
            return carry

        lax.fori_loop(0, NCH, chunk, 0)
        plsc.subcore_barrier()
        pltpu.sync_copy(acc_sh.at[sl], acc_h.at[c].at[sl])

    return k


def _dinv(deg_ref):
    deg = 1.0 + deg_ref[0][:, 0:1] + deg_ref[1][:, 0:1]
    return jnp.where(deg > 0, lax.rsqrt(deg), 0.0)


def _lin1_body(deg_ref, h0_ref, w1_ref, g1_ref):
    dinv = _dinv(deg_ref)
    hw = jnp.dot(h0_ref[...], w1_ref[...], preferred_element_type=jnp.float32)
    g1_ref[...] = dinv * hw


def _lin2_body(acc_ref, g1_ref, deg_ref, b1_ref, w2_ref, g2_ref):
    dinv = _dinv(deg_ref)
    a = acc_ref[0] + acc_ref[1] - g1_ref[...]
    pre = dinv * a + b1_ref[...]
    h1 = jnp.where(pre >= 0, pre, 0.01 * pre)
    hw = jnp.dot(h1, w2_ref[...], preferred_element_type=jnp.float32)
    g2_ref[...] = dinv * hw


def _final_body(acc_ref, g2_ref, deg_ref, b2_ref, out_ref):
    dinv = _dinv(deg_ref)
    z = dinv * (acc_ref[0] + acc_ref[1] - g2_ref[...]) + b2_ref[...]
    valid = lax.broadcasted_iota(jnp.int32, z.shape, 1) < 40
    zm = jnp.where(valid, z, -1e30)
    m = jnp.max(zm, axis=1, keepdims=True)
    e = jnp.where(valid, jnp.exp(zm - m), 0.0)
    ssum = jnp.sum(e, axis=1, keepdims=True)
    out_ref[...] = z - m - jnp.log(ssum)


def kernel(x, edge_index, edge_attr, emb, W1, b1, W2, b2):
    N = x.shape[0]
    E = edge_index.shape[1]
    V, D = emb.shape
    C = W2.shape[1]
    Dc2 = 48

    info = plsc.get_sparse_core_info()
    NC, NS = info.num_cores, info.num_subcores
    NW = NC * NS

    N_pad = -(-N // (NW * XCH)) * (NW * XCH)
    NCH = -(-E // (NW * CH))
    E_pad = NCH * NW * CH

    xi = jnp.concatenate([x[:, 0].astype(jnp.int32),
                          jnp.zeros((N_pad - N,), jnp.int32)])
    xi = xi.reshape(NW, (N_pad // NW) // XCH, XCH)
    epad = E_pad - E
    row3 = jnp.concatenate([edge_index[0].astype(jnp.int32),
                            jnp.zeros((epad,), jnp.int32)]).reshape(NW, NCH, CH)
    col3 = jnp.concatenate([edge_index[1].astype(jnp.int32),
                            jnp.zeros((epad,), jnp.int32)]).reshape(NW, NCH, CH)
    wp = jnp.concatenate([edge_attr, jnp.zeros((epad,), jnp.float32)])
    wfl = wp.reshape(NW, NCH * CH)
    wb = jnp.broadcast_to(wp[:, None], (E_pad, 16)).reshape(NW, NCH, CH, 16)
    z16 = jnp.zeros((N_pad, 16), jnp.float32)

    h0, deg2 = _emb_deg_kernel(V, D, N_pad, NCH, NC, NS)(emb, xi, col3, wb, z16)

    nblk = N_pad // 256
    deg_spec = pl.BlockSpec((NC, 256, 16), lambda i: (0, i, 0))
    g1 = pl.pallas_call(
        _lin1_body,
        grid=(nblk,),
        in_specs=[deg_spec,
                  pl.BlockSpec((256, D), lambda i: (i, 0)),
                  pl.BlockSpec((D, D), lambda i: (0, 0))],
        out_specs=pl.BlockSpec((256, D), lambda i: (i, 0)),
        out_shape=jax.ShapeDtypeStruct((N_pad, D), jnp.float32),
    )(deg2, h0, W1)

    acc1 = _agg_kernel(D, N_pad, NCH, NC, NS)(g1, row3, col3, wfl)

    W2p = jnp.pad(W2, ((0, 0), (0, Dc2 - C)))
    g2 = pl.pallas_call(
        _lin2_body,
        grid=(nblk,),
        in_specs=[pl.BlockSpec((NC, 256, D), lambda i: (0, i, 0)),
                  pl.BlockSpec((256, D), lambda i: (i, 0)),
                  deg_spec,
                  pl.BlockSpec((1, D), lambda i: (0, 0)),
                  pl.BlockSpec((D, Dc2), lambda i: (0, 0))],
        out_specs=pl.BlockSpec((256, Dc2), lambda i: (i, 0)),
        out_shape=jax.ShapeDtypeStruct((N_pad, Dc2), jnp.float32),
    )(acc1, g1, deg2, b1[None, :], W2p)

    acc2 = _agg_kernel(Dc2, N_pad, NCH, NC, NS)(g2, row3, col3, wfl)

    b2p = jnp.pad(b2, (0, Dc2 - C))
    out = pl.pallas_call(
        _final_body,
        grid=(nblk,),
        in_specs=[pl.BlockSpec((NC, 256, Dc2), lambda i: (0, i, 0)),
                  pl.BlockSpec((256, Dc2), lambda i: (i, 0)),
                  deg_spec,
                  pl.BlockSpec((1, Dc2), lambda i: (0, 0))],
        out_specs=pl.BlockSpec((256, Dc2), lambda i: (i, 0)),
        out_shape=jax.ShapeDtypeStruct((N_pad, Dc2), jnp.float32),
    )(acc2, g2, deg2, b2p[None, :])

    return out[:N, :C]


# speedup vs baseline: 8.5820x; 1.0019x over previous
import functools

import jax
import jax.numpy as jnp
from jax import lax
from jax.experimental import pallas as pl
from jax.experimental.pallas import tpu as pltpu
from jax.experimental.pallas import tpu_sc as plsc

CH = 128
XCH = 64

_SC_PARAMS = pltpu.CompilerParams(needs_layout_passes=False,
                                  use_tc_tiling_on_sc=False)


def _sc_mesh():
    return plsc.VectorSubcoreMesh(core_axis_name="c", subcore_axis_name="s")


def _emb_deg_kernel(V, D, N_pad, NCH, NC, NS):
    NW = NC * NS
    rows_w = N_pad // NW
    nx = rows_w // XCH
    rows_s = N_pad // NS

    @functools.partial(
        pl.kernel,
        out_type=[
            jax.ShapeDtypeStruct((N_pad, D), jnp.float32),
            jax.ShapeDtypeStruct((NC, N_pad, 16), jnp.float32),
        ],
        mesh=_sc_mesh(),
        compiler_params=_SC_PARAMS,
        scratch_types=[
            pltpu.VMEM((nx, XCH), jnp.int32),
            pltpu.VMEM((XCH, D), jnp.float32),
            pltpu.VMEM((NCH, CH), jnp.int32),
            pltpu.VMEM((CH, 16), jnp.float32),
            pltpu.VMEM_SHARED((N_pad, 16), jnp.float32),
        ],
    )
    def k(emb_h, xi_h, col_h, wb_h, z16_h, h0_h, deg_h,
          xi_v, rows_v, col_v, wrow_v, deg_sh):
        c = lax.axis_index("c")
        s = lax.axis_index("s")
        wid = s * NC + c
        sl = pl.ds(s * rows_s, rows_s)
        pltpu.sync_copy(z16_h.at[sl], deg_sh.at[sl])
        pltpu.sync_copy(xi_h.at[wid], xi_v)
        for j in range(nx):
            pltpu.sync_copy(emb_h.at[xi_v.at[j]], rows_v)
            pltpu.sync_copy(rows_v, h0_h.at[pl.ds(wid * rows_w + j * XCH, XCH)])
        pltpu.sync_copy(col_h.at[wid], col_v)
        plsc.subcore_barrier()

        def chunk(j, carry):
            pltpu.sync_copy(wb_h.at[wid].at[j], wrow_v)
            pltpu.sync_copy(wrow_v, deg_sh.at[col_v.at[j]], add=True)
            return carry

        lax.fori_loop(0, NCH, chunk, 0)
        plsc.subcore_barrier()
        pltpu.sync_copy(deg_sh.at[sl], deg_h.at[c].at[sl])

    return k


def _agg_kernel(Dc, N_pad, NP, NC, NS):
    rows_s = N_pad // NS

    @functools.partial(
        pl.kernel,
        out_type=jax.ShapeDtypeStruct((NC, N_pad, Dc), jnp.float32),
        mesh=_sc_mesh(),
        compiler_params=_SC_PARAMS,
        scratch_types=[
            pltpu.VMEM((2, 6, CH), jnp.int32),
            pltpu.VMEM((CH, Dc), jnp.float32),
            pltpu.VMEM((CH, Dc), jnp.float32),
            pltpu.VMEM_SHARED((N_pad, Dc), jnp.float32),
            pltpu.SemaphoreType.DMA((2,)),
            pltpu.SemaphoreType.DMA,
            pltpu.SemaphoreType.DMA,
            pltpu.SemaphoreType.DMA,
            pltpu.SemaphoreType.DMA,
        ],
    )
    def k(g_h, pk_h, acc_h, pk_v, buf0, buf1, acc_sh, pks, gs0, gs1, ss0, ss1):
        c = lax.axis_index("c")
        s = lax.axis_index("s")
        wid = s * NC + c
        sl = pl.ds(s * rows_s, rows_s)
        pltpu.sync_copy(g_h.at[sl], acc_sh.at[sl])
        plsc.subcore_barrier()

        def pkd(p, b):
            return pltpu.make_async_copy(pk_h.at[wid].at[p], pk_v.at[b],
                                         pks.at[b])

        def gat(b, par, buf, sem):
            return pltpu.make_async_copy(g_h.at[pk_v.at[b, par]], buf, sem)

        def sca(b, par, buf, sem):
            return pltpu.make_async_copy(buf, acc_sh.at[pk_v.at[b, 2 + par]],
                                         sem)

        def scale(b, par, buf):
            bs = lax.broadcast_in_dim(b, (16,), ())
            ws = lax.broadcast_in_dim(4 + par, (16,), ())

            def body(i, carry):
                wi = plsc.load_gather(
                    pk_v, [bs, ws, lax.broadcast_in_dim(i, (16,), ())])
                wb = plsc.bitcast(wi, jnp.float32)
                for kk in range(Dc // 16):
                    csl = pl.ds(kk * 16, 16)
                    buf[i, csl] = buf[i, csl] * wb
                return carry

            lax.fori_loop(0, CH, body, 0, unroll=2)

        pkd(0, 0).start()
        pkd(0, 0).wait()
        gat(0, 0, buf0, gs0).start()
        gat(0, 1, buf1, gs1).start()

        def pair(p, carry):
            b = p & 1
            nb = 1 - b

            @pl.when(p + 1 < NP)
            def _():
                pkd(p + 1, nb).start()

            gat(b, 0, buf0, gs0).wait()
            scale(b, 0, buf0)
            sca(b, 0, buf0, ss0).start(add=True)
            gat(b, 1, buf1, gs1).wait()
            scale(b, 1, buf1)
            sca(b, 1, buf1, ss1).start(add=True)

            @pl.when(p + 1 < NP)
            def _():
                pkd(p + 1, nb).wait()
                sca(b, 0, buf0, ss0).wait()
                gat(nb, 0, buf0, gs0).start()
                sca(b, 1, buf1, ss1).wait()
                gat(nb, 1, buf1, gs1).start()

            @pl.when(p + 1 >= NP)
            def _():
                sca(b, 0, buf0, ss0).wait()
                sca(b, 1, buf1, ss1).wait()

            return carry

        lax.fori_loop(0, NP, pair, 0)
        plsc.subcore_barrier()
        pltpu.sync_copy(acc_sh.at[sl], acc_h.at[c].at[sl])

    return k


def _dinv(deg_ref):
    deg = 1.0 + deg_ref[0][:, 0:1] + deg_ref[1][:, 0:1]
    return jnp.where(deg > 0, lax.rsqrt(deg), 0.0)


def _lin1_body(deg_ref, h0_ref, w1_ref, g1_ref):
    dinv = _dinv(deg_ref)
    hw = jnp.dot(h0_ref[...], w1_ref[...], preferred_element_type=jnp.float32)
    g1_ref[...] = dinv * hw


def _lin2_body(acc_ref, g1_ref, deg_ref, b1_ref, w2_ref, g2_ref):
    dinv = _dinv(deg_ref)
    a = acc_ref[0] + acc_ref[1] - g1_ref[...]
    pre = dinv * a + b1_ref[...]
    h1 = jnp.where(pre >= 0, pre, 0.01 * pre)
    hw = jnp.dot(h1, w2_ref[...], preferred_element_type=jnp.float32)
    g2_ref[...] = dinv * hw


def _final_body(acc_ref, g2_ref, deg_ref, b2_ref, out_ref):
    dinv = _dinv(deg_ref)
    z = dinv * (acc_ref[0] + acc_ref[1] - g2_ref[...]) + b2_ref[...]
    valid = lax.broadcasted_iota(jnp.int32, z.shape, 1) < 40
    zm = jnp.where(valid, z, -1e30)
    m = jnp.max(zm, axis=1, keepdims=True)
    e = jnp.where(valid, jnp.exp(zm - m), 0.0)
    ssum = jnp.sum(e, axis=1, keepdims=True)
    out_ref[...] = z - m - jnp.log(ssum)


def kernel(x, edge_index, edge_attr, emb, W1, b1, W2, b2):
    N = x.shape[0]
    E = edge_index.shape[1]
    V, D = emb.shape
    C = W2.shape[1]
    Dc2 = 48

    info = plsc.get_sparse_core_info()
    NC, NS = info.num_cores, info.num_subcores
    NW = NC * NS

    N_pad = -(-N // (NW * XCH)) * (NW * XCH)
    NCH = -(-E // (NW * CH))
    NCH += NCH % 2
    NP = NCH // 2
    E_pad = NCH * NW * CH

    xi = jnp.concatenate([x[:, 0].astype(jnp.int32),
                          jnp.zeros((N_pad - N,), jnp.int32)])
    xi = xi.reshape(NW, (N_pad // NW) // XCH, XCH)
    epad = E_pad - E
    rowp = jnp.concatenate([edge_index[0].astype(jnp.int32),
                            jnp.zeros((epad,), jnp.int32)])
    colp = jnp.concatenate([edge_index[1].astype(jnp.int32),
                            jnp.zeros((epad,), jnp.int32)])
    wp = jnp.concatenate([edge_attr, jnp.zeros((epad,), jnp.float32)])
    wbits = lax.bitcast_convert_type(wp, jnp.int32)
    pk = jnp.concatenate([rowp.reshape(NW, NP, 2, CH),
                          colp.reshape(NW, NP, 2, CH),
                          wbits.reshape(NW, NP, 2, CH)], axis=2)
    col3 = colp.reshape(NW, NCH, CH)
    wb = jnp.broadcast_to(wp[:, None], (E_pad, 16)).reshape(NW, NCH, CH, 16)
    z16 = jnp.zeros((N_pad, 16), jnp.float32)

    h0, deg2 = _emb_deg_kernel(V, D, N_pad, NCH, NC, NS)(emb, xi, col3, wb, z16)

    nblk = N_pad // 256
    deg_spec = pl.BlockSpec((NC, 256, 16), lambda i: (0, i, 0))
    g1 = pl.pallas_call(
        _lin1_body,
        grid=(nblk,),
        in_specs=[deg_spec,
                  pl.BlockSpec((256, D), lambda i: (i, 0)),
                  pl.BlockSpec((D, D), lambda i: (0, 0))],
        out_specs=pl.BlockSpec((256, D), lambda i: (i, 0)),
        out_shape=jax.ShapeDtypeStruct((N_pad, D), jnp.float32),
    )(deg2, h0, W1)

    acc1 = _agg_kernel(D, N_pad, NP, NC, NS)(g1, pk)

    W2p = jnp.pad(W2, ((0, 0), (0, Dc2 - C)))
    g2 = pl.pallas_call(
        _lin2_body,
        grid=(nblk,),
        in_specs=[pl.BlockSpec((NC, 256, D), lambda i: (0, i, 0)),
                  pl.BlockSpec((256, D), lambda i: (i, 0)),
                  deg_spec,
                  pl.BlockSpec((1, D), lambda i: (0, 0)),
                  pl.BlockSpec((D, Dc2), lambda i: (0, 0))],
        out_specs=pl.BlockSpec((256, Dc2), lambda i: (i, 0)),
        out_shape=jax.ShapeDtypeStruct((N_pad, Dc2), jnp.float32),
    )(acc1, g1, deg2, b1[None, :], W2p)

    acc2 = _agg_kernel(Dc2, N_pad, NP, NC, NS)(g2, pk)

    b2p = jnp.pad(b2, (0, Dc2 - C))
    out = pl.pallas_call(
        _final_body,
        grid=(nblk,),
        in_specs=[pl.BlockSpec((NC, 256, Dc2), lambda i: (0, i, 0)),
                  pl.BlockSpec((256, Dc2), lambda i: (i, 0)),
                  deg_spec,
                  pl.BlockSpec((1, Dc2), lambda i: (0, 0))],
        out_specs=pl.BlockSpec((256, Dc2), lambda i: (i, 0)),
        out_shape=jax.ShapeDtypeStruct((N_pad, Dc2), jnp.float32),
    )(acc2, g2, deg2, b2p[None, :])

    return out[:N, :C]
